# Initial kernel scaffold; baseline (speedup 1.0000x reference)
import jax
import jax.numpy as jnp
from jax.experimental import pallas as pl


def kernel(adjacency, input_feature, weight, bias):
    raise NotImplementedError("write your pallas kernel here")


# speedup vs baseline: 1.2590x; 1.2590x over previous
import functools

import jax
import jax.numpy as jnp
from jax.experimental import pallas as pl
from jax.experimental.pallas import tpu as pltpu

_BC = 384
_BR = 2000


def _scores_kernel(a_ref, x_ref, w_ref, b_ref, out_ref, acc_ref, *, n):
    c = pl.program_id(0)
    nc = pl.num_programs(0)

    @pl.when(c == 0)
    def _():
        acc_ref[...] = jnp.zeros_like(acc_ref)

    col = c * _BC + jax.lax.broadcasted_iota(jnp.int32, (1, _BC), 1)
    col_valid = col < n

    x = x_ref[...]
    support = jax.lax.dot_general(
        w_ref[...], x, (((0,), (1,)), ((), ())),
        preferred_element_type=jnp.float32)
    support = jnp.where(col_valid, support, 0.0)

    @pl.when(c < nc - 1)
    def _():
        acc_ref[...] += jax.lax.dot_general(
            support, a_ref[...], (((1,), (1,)), ((), ())),
            preferred_element_type=jnp.float32)

    @pl.when(c == nc - 1)
    def _():
        a = jnp.where(col_valid, a_ref[...], 0.0)
        acc_ref[...] += jax.lax.dot_general(
            support, a, (((1,), (1,)), ((), ())),
            preferred_element_type=jnp.float32)
        out_ref[...] = jnp.tanh(acc_ref[...] + b_ref[0, 0])


def _sort_key(f32val):
    b = jax.lax.bitcast_convert_type(f32val, jnp.int32)
    return b ^ (jax.lax.shift_right_arithmetic(b, 31) & jnp.int32(0x7FFFFFFF))


def _select_scale_kernel(s_row_ref, s_col_ref, x_ref, out_ref, sel_ref, *, n, k):
    r = pl.program_id(0)

    @pl.when(r == 0)
    def _():
        key = _sort_key(s_row_ref[...])
        idx = jax.lax.broadcasted_iota(jnp.int32, (1, n), 1)

        npos = jnp.sum((key >= 0).astype(jnp.int32))
        cand0 = jnp.where(npos >= k, jnp.int32(0), jnp.int32(-2147483648))

        def vbody(i, cand):
            test = cand | (jnp.int32(1) << (30 - i))
            cnt = jnp.sum((key >= test).astype(jnp.int32))
            return jnp.where(cnt >= k, test, cand)

        tkey = jax.lax.fori_loop(0, 31, vbody, cand0)

        eq = key == tkey
        cgt = jnp.sum((key > tkey).astype(jnp.int32))
        need = k - cgt

        def ibody(i, lohi):
            lo, hi = lohi
            mid = (lo + hi) // 2
            cnt = jnp.sum((eq & (idx < mid)).astype(jnp.int32))
            found = cnt >= need
            return (jnp.where(found, lo, mid), jnp.where(found, mid, hi))

        lo, hi = jax.lax.fori_loop(
            0, 14, ibody, (jnp.int32(0), jnp.int32(n)))
        sel_ref[0] = tkey
        sel_ref[1] = jnp.where(need > 0, hi, jnp.int32(0))

    tkey = sel_ref[0]
    cutoff = sel_ref[1]
    sc = s_col_ref[...]
    kc = _sort_key(sc)
    ridx = r * _BR + jax.lax.broadcasted_iota(jnp.int32, (_BR, 1), 0)
    keep = (kc > tkey) | ((kc == tkey) & (ridx < cutoff))
    coeff = jnp.where(keep, sc, 0.0)
    out_ref[...] = (x_ref[...] * coeff)[None]


def kernel(adjacency, input_feature, weight, bias):
    n, d = input_feature.shape
    k = max(int(0.5 * n), 1)
    nc = (n + _BC - 1) // _BC

    scores = pl.pallas_call(
        functools.partial(_scores_kernel, n=n),
        grid=(nc,),
        in_specs=[
            pl.BlockSpec((n, _BC), lambda c: (0, c)),
            pl.BlockSpec((_BC, d), lambda c: (c, 0)),
            pl.BlockSpec((d, 1), lambda c: (0, 0)),
            pl.BlockSpec((1, 1), lambda c: (0, 0)),
        ],
        out_specs=pl.BlockSpec((1, n), lambda c: (0, 0)),
        out_shape=jax.ShapeDtypeStruct((1, n), jnp.float32),
        scratch_shapes=[pltpu.VMEM((1, n), jnp.float32)],
        compiler_params=pltpu.CompilerParams(
            dimension_semantics=("arbitrary",)),
    )(adjacency, input_feature, weight, bias.reshape(1, 1))

    s_col = scores.reshape(n, 1)

    hidden = pl.pallas_call(
        functools.partial(_select_scale_kernel, n=n, k=k),
        grid=(n // _BR,),
        in_specs=[
            pl.BlockSpec((1, n), lambda r: (0, 0)),
            pl.BlockSpec((_BR, 1), lambda r: (r, 0)),
            pl.BlockSpec((_BR, d), lambda r: (r, 0)),
        ],
        out_specs=pl.BlockSpec((1, _BR, d), lambda r: (0, r, 0)),
        out_shape=jax.ShapeDtypeStruct((1, n, d), jnp.float32),
        scratch_shapes=[pltpu.SMEM((2,), jnp.int32)],
        compiler_params=pltpu.CompilerParams(
            dimension_semantics=("arbitrary",)),
    )(scores, s_col, input_feature)

    return hidden


# speedup vs baseline: 1.3199x; 1.0484x over previous
import functools

import jax
import jax.numpy as jnp
from jax.experimental import pallas as pl
from jax.experimental.pallas import tpu as pltpu

_BC = 384
_BR = 2000


def _sort_key(f32val):
    b = jax.lax.bitcast_convert_type(f32val, jnp.int32)
    return b ^ (jax.lax.shift_right_arithmetic(b, 31) & jnp.int32(0x7FFFFFFF))


def _fused_kernel(a_ref, xc_ref, xr_ref, w_ref, b_ref, out_ref,
                  acc_ref, scol_ref, sel_ref, *, n, k, nc):
    c = pl.program_id(0)

    @pl.when(c == 0)
    def _():
        acc_ref[...] = jnp.zeros_like(acc_ref)

    @pl.when(c < nc)
    def _():
        col = c * _BC + jax.lax.broadcasted_iota(jnp.int32, (1, _BC), 1)
        col_valid = col < n
        x = xc_ref[...]
        support = jax.lax.dot_general(
            w_ref[...], x, (((0,), (1,)), ((), ())),
            preferred_element_type=jnp.float32)
        support = jnp.where(col_valid, support, 0.0)

        @pl.when(c < nc - 1)
        def _():
            acc_ref[...] += jax.lax.dot_general(
                support, a_ref[...], (((1,), (1,)), ((), ())),
                preferred_element_type=jnp.float32)

        @pl.when(c == nc - 1)
        def _():
            a = jnp.where(col_valid, a_ref[...], 0.0)
            acc_ref[...] += jax.lax.dot_general(
                support, a, (((1,), (1,)), ((), ())),
                preferred_element_type=jnp.float32)

            score = jnp.tanh(acc_ref[...] + b_ref[0, 0])
            scol_ref[...] = jnp.reshape(score, (n, 1))

            key = _sort_key(score)
            idx = jax.lax.broadcasted_iota(jnp.int32, (1, n), 1)

            npos = jnp.sum((key >= 0).astype(jnp.int32))
            cand0 = jnp.where(npos >= k, jnp.int32(0), jnp.int32(-2147483648))

            def vbody(i, cand):
                test = cand | (jnp.int32(1) << (30 - i))
                cnt = jnp.sum((key >= test).astype(jnp.int32))
                return jnp.where(cnt >= k, test, cand)

            tkey = jax.lax.fori_loop(0, 31, vbody, cand0)

            eq = key == tkey
            cgt = jnp.sum((key > tkey).astype(jnp.int32))
            need = k - cgt

            def ibody(i, lohi):
                lo, hi = lohi
                mid = (lo + hi) // 2
                cnt = jnp.sum((eq & (idx < mid)).astype(jnp.int32))
                found = cnt >= need
                return (jnp.where(found, lo, mid), jnp.where(found, mid, hi))

            lo, hi = jax.lax.fori_loop(
                0, 14, ibody, (jnp.int32(0), jnp.int32(n)))
            sel_ref[0] = tkey
            sel_ref[1] = jnp.where(need > 0, hi, jnp.int32(0))

    @pl.when(c >= nc)
    def _():
        r = c - nc
        tkey = sel_ref[0]
        cutoff = sel_ref[1]
        sc = scol_ref[pl.ds(r * _BR, _BR), :]
        kc = _sort_key(sc)
        ridx = r * _BR + jax.lax.broadcasted_iota(jnp.int32, (_BR, 1), 0)
        keep = (kc > tkey) | ((kc == tkey) & (ridx < cutoff))
        coeff = jnp.where(keep, sc, 0.0)
        out_ref[...] = (xr_ref[...] * coeff)[None]


def kernel(adjacency, input_feature, weight, bias):
    n, d = input_feature.shape
    k = max(int(0.5 * n), 1)
    nc = (n + _BC - 1) // _BC
    nr = n // _BR

    hidden = pl.pallas_call(
        functools.partial(_fused_kernel, n=n, k=k, nc=nc),
        grid=(nc + nr,),
        in_specs=[
            pl.BlockSpec((n, _BC), lambda c: (0, jnp.minimum(c, nc - 1))),
            pl.BlockSpec((_BC, d), lambda c: (jnp.minimum(c, nc - 1), 0)),
            pl.BlockSpec((_BR, d), lambda c: (jnp.maximum(c - nc, 0), 0)),
            pl.BlockSpec((d, 1), lambda c: (0, 0)),
            pl.BlockSpec((1, 1), lambda c: (0, 0)),
        ],
        out_specs=pl.BlockSpec((1, _BR, d), lambda c: (0, jnp.maximum(c - nc, 0), 0)),
        out_shape=jax.ShapeDtypeStruct((1, n, d), jnp.float32),
        scratch_shapes=[
            pltpu.VMEM((1, n), jnp.float32),
            pltpu.VMEM((n, 1), jnp.float32),
            pltpu.SMEM((2,), jnp.int32),
        ],
        compiler_params=pltpu.CompilerParams(
            dimension_semantics=("arbitrary",)),
    )(adjacency, input_feature, input_feature, weight, bias.reshape(1, 1))

    return hidden


# speedup vs baseline: 1.3729x; 1.0401x over previous
import functools

import jax
import jax.numpy as jnp
from jax.experimental import pallas as pl
from jax.experimental.pallas import tpu as pltpu

_BC = 384
_BR = 1000


def _sort_key(f32val):
    b = jax.lax.bitcast_convert_type(f32val, jnp.int32)
    return b ^ (jax.lax.shift_right_arithmetic(b, 31) & jnp.int32(0x7FFFFFFF))


def _fused_kernel(a_ref, xc_ref, w_ref, b_ref, out_ref,
                  acc_ref, xres_ref, scol_ref, sel_ref, *, n, k, nc):
    c = pl.program_id(0)

    @pl.when(c == 0)
    def _():
        acc_ref[...] = jnp.zeros_like(acc_ref)

    @pl.when(c < nc)
    def _():
        col = c * _BC + jax.lax.broadcasted_iota(jnp.int32, (1, _BC), 1)
        col_valid = col < n
        x = xc_ref[...]
        xres_ref[pl.ds(c * _BC, _BC), :] = x
        support = jax.lax.dot_general(
            w_ref[...], x, (((0,), (1,)), ((), ())),
            preferred_element_type=jnp.float32)
        support = jnp.where(col_valid, support, 0.0)

        @pl.when(c < nc - 1)
        def _():
            acc_ref[...] += jax.lax.dot_general(
                support, a_ref[...], (((1,), (1,)), ((), ())),
                preferred_element_type=jnp.float32)

        @pl.when(c == nc - 1)
        def _():
            a = jnp.where(col_valid, a_ref[...], 0.0)
            acc_ref[...] += jax.lax.dot_general(
                support, a, (((1,), (1,)), ((), ())),
                preferred_element_type=jnp.float32)

            score = jnp.tanh(acc_ref[...] + b_ref[0, 0])
            scol_ref[...] = jnp.reshape(score, (n, 1))

            key = _sort_key(score)
            idx = jax.lax.broadcasted_iota(jnp.int32, (1, n), 1)

            npos = jnp.sum((key >= 0).astype(jnp.int32))
            cand0 = jnp.where(npos >= k, jnp.int32(0), jnp.int32(-2147483648))

            def vbody(i, cand):
                test = cand | (jnp.int32(1) << (30 - i))
                cnt = jnp.sum((key >= test).astype(jnp.int32))
                return jnp.where(cnt >= k, test, cand)

            tkey = jax.lax.fori_loop(0, 31, vbody, cand0)

            eq = key == tkey
            cgt = jnp.sum((key > tkey).astype(jnp.int32))
            need = k - cgt

            def ibody(i, lohi):
                lo, hi = lohi
                mid = (lo + hi) // 2
                cnt = jnp.sum((eq & (idx < mid)).astype(jnp.int32))
                found = cnt >= need
                return (jnp.where(found, lo, mid), jnp.where(found, mid, hi))

            lo, hi = jax.lax.fori_loop(
                0, 14, ibody, (jnp.int32(0), jnp.int32(n)))
            sel_ref[0] = tkey
            sel_ref[1] = jnp.where(need > 0, hi, jnp.int32(0))

    @pl.when(c >= nc)
    def _():
        r = c - nc
        tkey = sel_ref[0]
        cutoff = sel_ref[1]
        sc = scol_ref[pl.ds(r * _BR, _BR), :]
        kc = _sort_key(sc)
        ridx = r * _BR + jax.lax.broadcasted_iota(jnp.int32, (_BR, 1), 0)
        keep = (kc > tkey) | ((kc == tkey) & (ridx < cutoff))
        coeff = jnp.where(keep, sc, 0.0)
        out_ref[...] = (xres_ref[pl.ds(r * _BR, _BR), :] * coeff)[None]


def kernel(adjacency, input_feature, weight, bias):
    n, d = input_feature.shape
    k = max(int(0.5 * n), 1)
    nc = (n + _BC - 1) // _BC
    nr = n // _BR

    hidden = pl.pallas_call(
        functools.partial(_fused_kernel, n=n, k=k, nc=nc),
        grid=(nc + nr,),
        in_specs=[
            pl.BlockSpec((n, _BC), lambda c: (0, jnp.minimum(c, nc - 1))),
            pl.BlockSpec((_BC, d), lambda c: (jnp.minimum(c, nc - 1), 0)),
            pl.BlockSpec((d, 1), lambda c: (0, 0)),
            pl.BlockSpec((1, 1), lambda c: (0, 0)),
        ],
        out_specs=pl.BlockSpec((1, _BR, d), lambda c: (0, jnp.maximum(c - nc, 0), 0)),
        out_shape=jax.ShapeDtypeStruct((1, n, d), jnp.float32),
        scratch_shapes=[
            pltpu.VMEM((1, n), jnp.float32),
            pltpu.VMEM((nc * _BC, d), jnp.float32),
            pltpu.VMEM((n, 1), jnp.float32),
            pltpu.SMEM((2,), jnp.int32),
        ],
        compiler_params=pltpu.CompilerParams(
            dimension_semantics=("arbitrary",),
            vmem_limit_bytes=64 * 1024 * 1024),
    )(adjacency, input_feature, weight, bias.reshape(1, 1))

    return hidden


# speedup vs baseline: 1.3830x; 1.0074x over previous
import functools

import jax
import jax.numpy as jnp
from jax.experimental import pallas as pl
from jax.experimental.pallas import tpu as pltpu

_BC = 384
_BR = 1000


def _sort_key(f32val):
    b = jax.lax.bitcast_convert_type(f32val, jnp.int32)
    return b ^ (jax.lax.shift_right_arithmetic(b, 31) & jnp.int32(0x7FFFFFFF))


def _fused_kernel(at_ref, ab_ref, xc_ref, w_ref, b_ref, out_ref,
                  acc_ref, xres_ref, scol_ref, sel_ref, *, n, k, nc):
    c = pl.program_id(0)
    n2 = n // 2

    @pl.when(c == 0)
    def _():
        acc_ref[...] = jnp.zeros_like(acc_ref)

    @pl.when(c < nc)
    def _():
        col = c * _BC + jax.lax.broadcasted_iota(jnp.int32, (1, _BC), 1)
        col_valid = col < n
        x = xc_ref[...]
        xres_ref[pl.ds(c * _BC, _BC), :] = x
        support = jax.lax.dot_general(
            w_ref[...], x, (((0,), (1,)), ((), ())),
            preferred_element_type=jnp.float32)
        support = jnp.where(col_valid, support, 0.0)

        @pl.when(c < nc - 1)
        def _():
            acc_ref[0:1, :] += jax.lax.dot_general(
                support, at_ref[...], (((1,), (1,)), ((), ())),
                preferred_element_type=jnp.float32)
            acc_ref[1:2, :] += jax.lax.dot_general(
                support, ab_ref[...], (((1,), (1,)), ((), ())),
                preferred_element_type=jnp.float32)

        @pl.when(c == nc - 1)
        def _():
            at = jnp.where(col_valid, at_ref[...], 0.0)
            ab = jnp.where(col_valid, ab_ref[...], 0.0)
            acc_ref[0:1, :] += jax.lax.dot_general(
                support, at, (((1,), (1,)), ((), ())),
                preferred_element_type=jnp.float32)
            acc_ref[1:2, :] += jax.lax.dot_general(
                support, ab, (((1,), (1,)), ((), ())),
                preferred_element_type=jnp.float32)

            score = jnp.tanh(acc_ref[...] + b_ref[0, 0])
            scol_ref[pl.ds(0, n2), :] = jnp.reshape(score[0:1, :], (n2, 1))
            scol_ref[pl.ds(n2, n2), :] = jnp.reshape(score[1:2, :], (n2, 1))

            key = _sort_key(score)
            idx = (jax.lax.broadcasted_iota(jnp.int32, (2, n2), 0) * n2
                   + jax.lax.broadcasted_iota(jnp.int32, (2, n2), 1))

            npos = jnp.sum((key >= 0).astype(jnp.int32))
            cand0 = jnp.where(npos >= k, jnp.int32(0), jnp.int32(-2147483648))

            def vbody(i, cand):
                test = cand | (jnp.int32(1) << (30 - i))
                cnt = jnp.sum((key >= test).astype(jnp.int32))
                return jnp.where(cnt >= k, test, cand)

            tkey = jax.lax.fori_loop(0, 31, vbody, cand0)

            eq = key == tkey
            cgt = jnp.sum((key > tkey).astype(jnp.int32))
            need = k - cgt

            def ibody(i, lohi):
                lo, hi = lohi
                mid = (lo + hi) // 2
                cnt = jnp.sum((eq & (idx < mid)).astype(jnp.int32))
                found = cnt >= need
                return (jnp.where(found, lo, mid), jnp.where(found, mid, hi))

            lo, hi = jax.lax.fori_loop(
                0, 14, ibody, (jnp.int32(0), jnp.int32(n)))
            sel_ref[0] = tkey
            sel_ref[1] = jnp.where(need > 0, hi, jnp.int32(0))

    @pl.when(c >= nc)
    def _():
        r = c - nc
        tkey = sel_ref[0]
        cutoff = sel_ref[1]
        sc = scol_ref[pl.ds(r * _BR, _BR), :]
        kc = _sort_key(sc)
        ridx = r * _BR + jax.lax.broadcasted_iota(jnp.int32, (_BR, 1), 0)
        keep = (kc > tkey) | ((kc == tkey) & (ridx < cutoff))
        coeff = jnp.where(keep, sc, 0.0)
        out_ref[...] = (xres_ref[pl.ds(r * _BR, _BR), :] * coeff)[None]


def kernel(adjacency, input_feature, weight, bias):
    n, d = input_feature.shape
    k = max(int(0.5 * n), 1)
    nc = (n + _BC - 1) // _BC
    nr = n // _BR

    hidden = pl.pallas_call(
        functools.partial(_fused_kernel, n=n, k=k, nc=nc),
        grid=(nc + nr,),
        in_specs=[
            pl.BlockSpec((n // 2, _BC), lambda c: (0, jnp.minimum(c, nc - 1))),
            pl.BlockSpec((n // 2, _BC), lambda c: (1, jnp.minimum(c, nc - 1))),
            pl.BlockSpec((_BC, d), lambda c: (jnp.minimum(c, nc - 1), 0)),
            pl.BlockSpec((d, 1), lambda c: (0, 0)),
            pl.BlockSpec((1, 1), lambda c: (0, 0)),
        ],
        out_specs=pl.BlockSpec((1, _BR, d), lambda c: (0, jnp.maximum(c - nc, 0), 0)),
        out_shape=jax.ShapeDtypeStruct((1, n, d), jnp.float32),
        scratch_shapes=[
            pltpu.VMEM((2, n // 2), jnp.float32),
            pltpu.VMEM((nc * _BC, d), jnp.float32),
            pltpu.VMEM((n, 1), jnp.float32),
            pltpu.SMEM((2,), jnp.int32),
        ],
        compiler_params=pltpu.CompilerParams(
            dimension_semantics=("arbitrary",),
            vmem_limit_bytes=64 * 1024 * 1024),
    )(adjacency, adjacency, input_feature, weight, bias.reshape(1, 1))

    return hidden
